# baseline (speedup 1.0000x reference)
import jax
import jax.numpy as jnp
from jax import lax
from jax.experimental import pallas as pl
from jax.experimental.pallas import tpu as pltpu
from jax.experimental.pallas import tpu_sc as plsc

_VOC = 16384
_HID = 64
_B = 1024
_NC = 2
_NS = 16
_NW = _NC * _NS
_MB = 128
_NMB = 68
_EW = _MB * _NMB
_NNZ_PAD = _NW * _EW
_RPT = _VOC // _NS
_EPS = 1e-5
_NB = 6
_PF = 3


def _sc_body(row_hbm, col_hbm, vals_hbm, w_hbm, p_hbm, d_hbm,
             row_v, col_v, vals_v, rows_buf, zrow, acc_s, deg_s,
             sem_g, sem_s, sem_d, sem_t):
    c = lax.axis_index("c")
    s = lax.axis_index("s")
    w = c * _NS + s

    init_scope = jax.named_scope("sc_init")
    init_scope.__enter__()
    pltpu.sync_copy(vals_hbm.at[w], vals_v)

    zv = jnp.zeros((16,), jnp.float32)

    def zb_body(i, carry):
        rows_buf[0, i // 4, pl.ds((i % 4) * 16, 16)] = zv
        return carry
    lax.fori_loop(0, _MB * _HID // 16, zb_body, 0)

    def zr_body(i, carry):
        zrow[pl.ds(i * 16, 16)] = zv
        return carry
    lax.fori_loop(0, _RPT // 16, zr_body, 0)

    base = s * _RPT
    for i in range(_RPT // _MB):
        pltpu.sync_copy(rows_buf.at[0], acc_s.at[pl.ds(base + i * _MB, _MB)])
    pltpu.sync_copy(zrow, deg_s.at[pl.ds(base, _RPT)])
    plsc.subcore_barrier()
    init_scope.__exit__(None, None, None)
    main_scope = jax.named_scope("sc_mainloop")
    main_scope.__enter__()

    def start_stage(j, t):
        pltpu.async_copy(row_hbm.at[w, j], row_v.at[t], sem_t.at[t])
        pltpu.async_copy(col_hbm.at[w, j], col_v.at[t], sem_t.at[t])

    def wait_stage(t):
        for _ in range(2):
            pltpu.make_async_copy(row_hbm.at[w, 0], row_v.at[t],
                                  sem_t.at[t]).wait()

    def start_gather(j, b):
        pltpu.async_copy(w_hbm.at[col_v.at[b]], rows_buf.at[b], sem_g.at[b])

    def wait_gather(b):
        pltpu.make_async_copy(w_hbm.at[col_v.at[0]], rows_buf.at[b],
                              sem_g.at[b]).wait()

    def start_scatter(j, b):
        pltpu.async_copy(rows_buf.at[b], acc_s.at[row_v.at[b]], sem_s.at[b],
                         add=True)
        pltpu.async_copy(vals_v.at[pl.ds(j * _MB, _MB)],
                         deg_s.at[row_v.at[b]], sem_d.at[b], add=True)

    def wait_scatter(b):
        pltpu.make_async_copy(rows_buf.at[b], acc_s.at[row_v.at[0]],
                              sem_s.at[b]).wait()
        pltpu.make_async_copy(vals_v.at[pl.ds(0, _MB)],
                              deg_s.at[row_v.at[0]], sem_d.at[b]).wait()

    def scale(j, b):
        jj = j * _MB

        def blk_body(eb, carry2):
            vv = vals_v[pl.ds(jj + eb * 16, 16)]
            for l in range(16):
                val = jnp.full((16,), vv[l], jnp.float32)
                e = eb * 16 + l
                for q in range(4):
                    sl = (b, e, pl.ds(q * 16, 16))
                    rows_buf[sl] = rows_buf[sl] * val
            return carry2
        lax.fori_loop(0, _MB // 16, blk_body, 0)

    def run_iter(j, drain, stage, gather):
        b = j % _NB
        if drain:
            wait_scatter((j + _PF + 1) % _NB)
        if stage:
            start_stage(j + _PF + 1, (j + _PF + 1) % _NB)
        if gather:
            wait_stage((j + _PF) % _NB)
            start_gather(j + _PF, (j + _PF) % _NB)
        wait_gather(b)
        scale(j, b)
        start_scatter(j, b)

    for j in range(_PF + 1):
        start_stage(j, j)
    for j in range(_PF):
        wait_stage(j)
        start_gather(j, j)
    for j in range(_NB):
        run_iter(j, drain=(j >= _NB - _PF - 1), stage=True, gather=True)

    def mb_body(j, carry):
        run_iter(j, drain=True, stage=True, gather=True)
        return carry
    lax.fori_loop(_NB, _NMB - _PF - 1, mb_body, 0)

    for j in range(_NMB - _PF - 1, _NMB):
        run_iter(j, drain=True, stage=(j + _PF + 1 < _NMB),
                 gather=(j + _PF < _NMB))
    for j in range(_NMB - _NB + _PF + 1, _NMB):
        wait_scatter(j % _NB)

    main_scope.__exit__(None, None, None)
    out_scope = jax.named_scope("sc_readout")
    out_scope.__enter__()
    plsc.subcore_barrier()
    pltpu.sync_copy(acc_s.at[pl.ds(base, _RPT)],
                    p_hbm.at[c, pl.ds(base, _RPT)])
    pltpu.sync_copy(deg_s.at[pl.ds(base, _RPT)],
                    d_hbm.at[c, pl.ds(base, _RPT)])
    out_scope.__exit__(None, None, None)


def _sc_partials(row, col, vals, w_neigh):
    mesh = plsc.VectorSubcoreMesh(core_axis_name="c", subcore_axis_name="s")
    return pl.kernel(
        _sc_body,
        out_type=[
            jax.ShapeDtypeStruct((_NC, _VOC, _HID), jnp.float32),
            jax.ShapeDtypeStruct((_NC, _VOC), jnp.float32),
        ],
        mesh=mesh,
        compiler_params=pltpu.CompilerParams(use_tc_tiling_on_sc=False),
        scratch_types=[
            pltpu.VMEM((_NB, _MB), jnp.int32),
            pltpu.VMEM((_NB, _MB), jnp.int32),
            pltpu.VMEM((_EW,), jnp.float32),
            pltpu.VMEM((_NB, _MB, _HID), jnp.float32),
            pltpu.VMEM((_RPT,), jnp.float32),
            pltpu.VMEM_SHARED((_VOC, _HID), jnp.float32),
            pltpu.VMEM_SHARED((_VOC,), jnp.float32),
            pltpu.SemaphoreType.DMA((_NB,)),
            pltpu.SemaphoreType.DMA((_NB,)),
            pltpu.SemaphoreType.DMA((_NB,)),
            pltpu.SemaphoreType.DMA((_NB,)),
        ],
    )(row, col, vals, w_neigh)


_KB = 2048
_NKB = _VOC // _KB


def _tc_body(x_ref, p_ref, d_ref, ws_ref, g_ref, b_ref, fw_ref, fb_ref,
             o_ref, acc_ref):
    k = pl.program_id(0)

    @pl.when(k == 0)
    def _():
        acc_ref[...] = jnp.zeros_like(acc_ref)

    neigh = p_ref[0] + p_ref[1]
    deg = jnp.maximum(jnp.sum(d_ref[...], axis=1, keepdims=True), 1.0)
    h = jnp.maximum(ws_ref[...] + neigh / deg, 0.0)
    mu = jnp.mean(h, axis=1, keepdims=True)
    hc = h - mu
    var = jnp.mean(hc * hc, axis=1, keepdims=True)
    hn = hc * lax.rsqrt(var + _EPS) * g_ref[...] + b_ref[...]
    acc_ref[...] += jnp.dot(x_ref[...], hn,
                            preferred_element_type=jnp.float32)

    @pl.when(k == _NKB - 1)
    def _():
        o_ref[...] = jnp.dot(acc_ref[...], fw_ref[...],
                             preferred_element_type=jnp.float32) + fb_ref[...]


def _tc_call(x, p, dt, w_self, g, b, fw, fb):
    return pl.pallas_call(
        _tc_body,
        grid=(_NKB,),
        in_specs=[
            pl.BlockSpec((_B, _KB), lambda k: (0, k)),
            pl.BlockSpec((_NC, _KB, _HID), lambda k: (0, k, 0)),
            pl.BlockSpec((_KB, _NC), lambda k: (k, 0)),
            pl.BlockSpec((_KB, _HID), lambda k: (k, 0)),
            pl.BlockSpec((1, _HID), lambda k: (0, 0)),
            pl.BlockSpec((1, _HID), lambda k: (0, 0)),
            pl.BlockSpec((_HID, _HID), lambda k: (0, 0)),
            pl.BlockSpec((1, _HID), lambda k: (0, 0)),
        ],
        out_specs=pl.BlockSpec((_B, _HID), lambda k: (0, 0)),
        out_shape=jax.ShapeDtypeStruct((_B, _HID), jnp.float32),
        scratch_shapes=[pltpu.VMEM((_B, _HID), jnp.float32)],
    )(x, p, dt, w_self, g, b, fw, fb)


def kernel(adj_indices, adj_values, X_dv, W_self, W_neigh, ln_gamma, ln_beta,
           fc_W, fc_b):
    nnz = adj_values.shape[0]
    pad = _NNZ_PAD - nnz
    zi = jnp.zeros((pad,), jnp.int32)
    row = jnp.concatenate([adj_indices[0], zi]).reshape(_NW, _NMB, _MB)
    col = jnp.concatenate([adj_indices[1], zi]).reshape(_NW, _NMB, _MB)
    vals = jnp.concatenate([adj_values, jnp.zeros((pad,), jnp.float32)])
    vals = vals.reshape(_NW, _EW)
    p, d = _sc_partials(row, col, vals, W_neigh)
    return _tc_call(X_dv, p, d.T, W_self, ln_gamma.reshape(1, _HID),
                    ln_beta.reshape(1, _HID), fc_W, fc_b.reshape(1, _HID))


# speedup vs baseline: 1.5749x; 1.5749x over previous
import jax
import jax.numpy as jnp
from jax import lax
from jax.experimental import pallas as pl
from jax.experimental.pallas import tpu as pltpu
from jax.experimental.pallas import tpu_sc as plsc

_VOC = 16384
_HID = 64
_B = 1024
_NC = 2
_NS = 16
_NW = _NC * _NS
_MB = 128
_NMB = 68
_EW = _MB * _NMB
_NNZ_PAD = _NW * _EW
_RPT = _VOC // _NS
_EPS = 1e-5
_NB = 6
_PF = 3


def _sc_body(row_hbm, col_hbm, vals_hbm, w_hbm, p_hbm, d_hbm,
             row_v, col_v, vals_v, rows_buf, zrow, acc_s, deg_s,
             sem_g, sem_s, sem_d, sem_t):
    c = lax.axis_index("c")
    s = lax.axis_index("s")
    w = c * _NS + s

    init_scope = jax.named_scope("sc_init")
    init_scope.__enter__()
    pltpu.sync_copy(vals_hbm.at[w], vals_v)

    zv = jnp.zeros((16,), jnp.float32)

    def zb_body(i, carry):
        rows_buf[0, i // 4, pl.ds((i % 4) * 16, 16)] = zv
        return carry
    lax.fori_loop(0, _MB * _HID // 16, zb_body, 0)

    def zr_body(i, carry):
        zrow[pl.ds(i * 16, 16)] = zv
        return carry
    lax.fori_loop(0, _RPT // 16, zr_body, 0)

    base = s * _RPT
    for i in range(_RPT // _MB):
        pltpu.sync_copy(rows_buf.at[0], acc_s.at[pl.ds(base + i * _MB, _MB)])
    pltpu.sync_copy(zrow, deg_s.at[pl.ds(base, _RPT)])
    plsc.subcore_barrier()
    init_scope.__exit__(None, None, None)
    main_scope = jax.named_scope("sc_mainloop")
    main_scope.__enter__()

    def start_stage(j, t):
        pltpu.async_copy(row_hbm.at[w, j], row_v.at[t], sem_t.at[t])
        pltpu.async_copy(col_hbm.at[w, j], col_v.at[t], sem_t.at[t])

    def wait_stage(t):
        for _ in range(2):
            pltpu.make_async_copy(row_hbm.at[w, 0], row_v.at[t],
                                  sem_t.at[t]).wait()

    def start_gather(j, b):
        pltpu.async_copy(w_hbm.at[col_v.at[b]], rows_buf.at[b], sem_g.at[b])

    def wait_gather(b):
        pltpu.make_async_copy(w_hbm.at[col_v.at[0]], rows_buf.at[b],
                              sem_g.at[b]).wait()

    def start_scatter(j, b):
        pltpu.async_copy(rows_buf.at[b], acc_s.at[row_v.at[b]], sem_s.at[b],
                         add=True)
        pltpu.async_copy(vals_v.at[pl.ds(j * _MB, _MB)],
                         deg_s.at[row_v.at[b]], sem_d.at[b], add=True)

    def wait_scatter(b):
        pltpu.make_async_copy(rows_buf.at[b], acc_s.at[row_v.at[0]],
                              sem_s.at[b]).wait()
        pltpu.make_async_copy(vals_v.at[pl.ds(0, _MB)],
                              deg_s.at[row_v.at[0]], sem_d.at[b]).wait()

    def scale(j, b):
        jj = j * _MB

        def blk_body(eb, carry2):
            vv = vals_v[pl.ds(jj + eb * 16, 16)]
            for l in range(16):
                val = jnp.full((16,), vv[l], jnp.float32)
                e = eb * 16 + l
                for q in range(4):
                    sl = (b, e, pl.ds(q * 16, 16))
                    rows_buf[sl] = rows_buf[sl] * val
            return carry2
        lax.fori_loop(0, _MB // 16, blk_body, 0)

    def run_iter(j, drain, stage, gather):
        b = j % _NB
        if drain:
            wait_scatter((j + _PF + 1) % _NB)
        if stage:
            start_stage(j + _PF + 1, (j + _PF + 1) % _NB)
        if gather:
            wait_stage((j + _PF) % _NB)
            start_gather(j + _PF, (j + _PF) % _NB)
        wait_gather(b)
        scale(j, b)
        start_scatter(j, b)

    for j in range(_PF + 1):
        start_stage(j, j)
    for j in range(_PF):
        wait_stage(j)
        start_gather(j, j)
    for j in range(_NB):
        run_iter(j, drain=(j >= _NB - _PF - 1), stage=True, gather=True)

    def mb_body(j, carry):
        run_iter(j, drain=True, stage=True, gather=True)
        return carry
    lax.fori_loop(_NB, _NMB - _PF - 1, mb_body, 0)

    for j in range(_NMB - _PF - 1, _NMB):
        run_iter(j, drain=True, stage=(j + _PF + 1 < _NMB),
                 gather=(j + _PF < _NMB))
    for j in range(_NMB - _NB + _PF + 1, _NMB):
        wait_scatter(j % _NB)

    main_scope.__exit__(None, None, None)
    out_scope = jax.named_scope("sc_readout")
    out_scope.__enter__()
    plsc.subcore_barrier()
    pltpu.sync_copy(acc_s.at[pl.ds(base, _RPT)],
                    p_hbm.at[c, pl.ds(base, _RPT)])
    pltpu.sync_copy(deg_s.at[pl.ds(base, _RPT)],
                    d_hbm.at[c, pl.ds(base, _RPT)])
    out_scope.__exit__(None, None, None)


def _sc_partials(row, col, vals, w_neigh):
    mesh = plsc.VectorSubcoreMesh(core_axis_name="c", subcore_axis_name="s")
    return pl.kernel(
        _sc_body,
        out_type=[
            jax.ShapeDtypeStruct((_NC, _VOC, _HID), jnp.float32),
            jax.ShapeDtypeStruct((_NC, _VOC), jnp.float32),
        ],
        mesh=mesh,
        compiler_params=pltpu.CompilerParams(use_tc_tiling_on_sc=False),
        scratch_types=[
            pltpu.VMEM((_NB, _MB), jnp.int32),
            pltpu.VMEM((_NB, _MB), jnp.int32),
            pltpu.VMEM((_EW,), jnp.float32),
            pltpu.VMEM((_NB, _MB, _HID), jnp.float32),
            pltpu.VMEM((_RPT,), jnp.float32),
            pltpu.VMEM_SHARED((_VOC, _HID), jnp.float32),
            pltpu.VMEM_SHARED((_VOC,), jnp.float32),
            pltpu.SemaphoreType.DMA((_NB,)),
            pltpu.SemaphoreType.DMA((_NB,)),
            pltpu.SemaphoreType.DMA((_NB,)),
            pltpu.SemaphoreType.DMA((_NB,)),
        ],
    )(row, col, vals, w_neigh)


_KB = 2048
_NKB = _VOC // _KB


def _tc_body(x_ref, p_ref, d_ref, ws_ref, g_ref, b_ref, fw_ref, fb_ref,
             o_ref, acc_ref):
    k = pl.program_id(0)

    @pl.when(k == 0)
    def _():
        acc_ref[...] = jnp.zeros_like(acc_ref)

    neigh = p_ref[0] + p_ref[1]
    deg = jnp.maximum(jnp.sum(d_ref[...], axis=1, keepdims=True), 1.0)
    h = jnp.maximum(ws_ref[...] + neigh / deg, 0.0)
    mu = jnp.mean(h, axis=1, keepdims=True)
    hc = h - mu
    var = jnp.mean(hc * hc, axis=1, keepdims=True)
    hn = hc * lax.rsqrt(var + _EPS) * g_ref[...] + b_ref[...]
    acc_ref[...] += jnp.dot(x_ref[...], hn,
                            preferred_element_type=jnp.float32)

    @pl.when(k == _NKB - 1)
    def _():
        o_ref[...] = jnp.dot(acc_ref[...], fw_ref[...],
                             preferred_element_type=jnp.float32) + fb_ref[...]


def _tc_call(x, p, dt, w_self, g, b, fw, fb):
    return pl.pallas_call(
        _tc_body,
        grid=(_NKB,),
        in_specs=[
            pl.BlockSpec((_B, _KB), lambda k: (0, k)),
            pl.BlockSpec((_NC, _KB, _HID), lambda k: (0, k, 0)),
            pl.BlockSpec((_KB, _NC), lambda k: (k, 0)),
            pl.BlockSpec((_KB, _HID), lambda k: (k, 0)),
            pl.BlockSpec((1, _HID), lambda k: (0, 0)),
            pl.BlockSpec((1, _HID), lambda k: (0, 0)),
            pl.BlockSpec((_HID, _HID), lambda k: (0, 0)),
            pl.BlockSpec((1, _HID), lambda k: (0, 0)),
        ],
        out_specs=pl.BlockSpec((_B, _HID), lambda k: (0, 0)),
        out_shape=jax.ShapeDtypeStruct((_B, _HID), jnp.float32),
        scratch_shapes=[pltpu.VMEM((_B, _HID), jnp.float32)],
    )(x, p, dt, w_self, g, b, fw, fb)


def kernel(adj_indices, adj_values, X_dv, W_self, W_neigh, ln_gamma, ln_beta,
           fc_W, fc_b):
    nnz = adj_values.shape[0]
    pad = _NNZ_PAD - nnz
    zi = jnp.arange(pad, dtype=jnp.int32) % _VOC
    row = jnp.concatenate([adj_indices[0], zi]).reshape(_NW, _NMB, _MB)
    col = jnp.concatenate([adj_indices[1], zi]).reshape(_NW, _NMB, _MB)
    vals = jnp.concatenate([adj_values, jnp.zeros((pad,), jnp.float32)])
    vals = vals.reshape(_NW, _EW)
    p, d = _sc_partials(row, col, vals, W_neigh)
    return _tc_call(X_dv, p, d.T, W_self, ln_gamma.reshape(1, _HID),
                    ln_beta.reshape(1, _HID), fc_W, fc_b.reshape(1, _HID))


# speedup vs baseline: 1.6516x; 1.0487x over previous
import jax
import jax.numpy as jnp
from jax import lax
from jax.experimental import pallas as pl
from jax.experimental.pallas import tpu as pltpu
from jax.experimental.pallas import tpu_sc as plsc

_VOC = 16384
_HID = 64
_B = 1024
_NC = 2
_NS = 16
_NW = _NC * _NS
_MB = 128
_NMB = 68
_EW = _MB * _NMB
_NNZ_PAD = _NW * _EW
_RPT = _VOC // _NS
_EPS = 1e-5
_NB = 6
_PF = 3


def _sc_body(idx_hbm, vals_hbm, w_hbm, p_hbm, d_hbm,
             row_v, col_v, vals_v, rows_buf, zrow, acc_s, deg_s,
             sem_g, sem_s, sem_d, sem_t):
    c = lax.axis_index("c")
    s = lax.axis_index("s")
    w = c * _NS + s

    init_scope = jax.named_scope("sc_init")
    init_scope.__enter__()
    pltpu.sync_copy(vals_hbm.at[pl.ds(w * _EW, _EW)], vals_v)

    zv = jnp.zeros((16,), jnp.float32)

    def zb_body(i, carry):
        rows_buf[0, i // 4, pl.ds((i % 4) * 16, 16)] = zv
        return carry
    lax.fori_loop(0, _MB * _HID // 16, zb_body, 0)

    def zr_body(i, carry):
        zrow[pl.ds(i * 16, 16)] = zv
        return carry
    lax.fori_loop(0, _RPT // 16, zr_body, 0)

    base = s * _RPT
    for i in range(_RPT // _MB):
        pltpu.sync_copy(rows_buf.at[0], acc_s.at[pl.ds(base + i * _MB, _MB)])
    pltpu.sync_copy(zrow, deg_s.at[pl.ds(base, _RPT)])
    plsc.subcore_barrier()
    init_scope.__exit__(None, None, None)
    main_scope = jax.named_scope("sc_mainloop")
    main_scope.__enter__()

    def start_stage(j, t):
        off = w * _EW + j * _MB
        pltpu.async_copy(idx_hbm.at[0, pl.ds(off, _MB)], row_v.at[t],
                         sem_t.at[t])
        pltpu.async_copy(idx_hbm.at[1, pl.ds(off, _MB)], col_v.at[t],
                         sem_t.at[t])

    def wait_stage(t):
        for _ in range(2):
            pltpu.make_async_copy(idx_hbm.at[0, pl.ds(0, _MB)], row_v.at[t],
                                  sem_t.at[t]).wait()

    def start_gather(j, b):
        pltpu.async_copy(w_hbm.at[col_v.at[b]], rows_buf.at[b], sem_g.at[b])

    def wait_gather(b):
        pltpu.make_async_copy(w_hbm.at[col_v.at[0]], rows_buf.at[b],
                              sem_g.at[b]).wait()

    def start_scatter(j, b):
        pltpu.async_copy(rows_buf.at[b], acc_s.at[row_v.at[b]], sem_s.at[b],
                         add=True)
        pltpu.async_copy(vals_v.at[pl.ds(j * _MB, _MB)],
                         deg_s.at[row_v.at[b]], sem_d.at[b], add=True)

    def wait_scatter(b):
        pltpu.make_async_copy(rows_buf.at[b], acc_s.at[row_v.at[0]],
                              sem_s.at[b]).wait()
        pltpu.make_async_copy(vals_v.at[pl.ds(0, _MB)],
                              deg_s.at[row_v.at[0]], sem_d.at[b]).wait()

    def scale(j, b):
        jj = j * _MB

        def blk_body(eb, carry2):
            vv = vals_v[pl.ds(jj + eb * 16, 16)]
            for l in range(16):
                val = jnp.full((16,), vv[l], jnp.float32)
                e = eb * 16 + l
                for q in range(4):
                    sl = (b, e, pl.ds(q * 16, 16))
                    rows_buf[sl] = rows_buf[sl] * val
            return carry2
        lax.fori_loop(0, _MB // 16, blk_body, 0)

    def run_iter(j, drain, stage, gather):
        b = j % _NB
        if drain:
            wait_scatter((j + _PF + 1) % _NB)
        if stage:
            start_stage(j + _PF + 1, (j + _PF + 1) % _NB)
        if gather:
            wait_stage((j + _PF) % _NB)
            start_gather(j + _PF, (j + _PF) % _NB)
        wait_gather(b)
        scale(j, b)
        start_scatter(j, b)

    for j in range(_PF + 1):
        start_stage(j, j)
    for j in range(_PF):
        wait_stage(j)
        start_gather(j, j)
    for j in range(_NB):
        run_iter(j, drain=(j >= _NB - _PF - 1), stage=True, gather=True)

    def mb_body(j, carry):
        run_iter(j, drain=True, stage=True, gather=True)
        return carry
    lax.fori_loop(_NB, _NMB - _PF - 1, mb_body, 0)

    for j in range(_NMB - _PF - 1, _NMB):
        run_iter(j, drain=True, stage=(j + _PF + 1 < _NMB),
                 gather=(j + _PF < _NMB))
    for j in range(_NMB - _NB + _PF + 1, _NMB):
        wait_scatter(j % _NB)

    main_scope.__exit__(None, None, None)
    out_scope = jax.named_scope("sc_readout")
    out_scope.__enter__()
    plsc.subcore_barrier()
    pltpu.sync_copy(acc_s.at[pl.ds(base, _RPT)],
                    p_hbm.at[c, pl.ds(base, _RPT)])
    pltpu.sync_copy(deg_s.at[pl.ds(base, _RPT)],
                    d_hbm.at[c, pl.ds(base, _RPT)])
    out_scope.__exit__(None, None, None)


def _sc_partials(idx, vals, w_neigh):
    mesh = plsc.VectorSubcoreMesh(core_axis_name="c", subcore_axis_name="s")
    return pl.kernel(
        _sc_body,
        out_type=[
            jax.ShapeDtypeStruct((_NC, _VOC, _HID), jnp.float32),
            jax.ShapeDtypeStruct((_NC, _VOC), jnp.float32),
        ],
        mesh=mesh,
        compiler_params=pltpu.CompilerParams(use_tc_tiling_on_sc=False),
        scratch_types=[
            pltpu.VMEM((_NB, _MB), jnp.int32),
            pltpu.VMEM((_NB, _MB), jnp.int32),
            pltpu.VMEM((_EW,), jnp.float32),
            pltpu.VMEM((_NB, _MB, _HID), jnp.float32),
            pltpu.VMEM((_RPT,), jnp.float32),
            pltpu.VMEM_SHARED((_VOC, _HID), jnp.float32),
            pltpu.VMEM_SHARED((_VOC,), jnp.float32),
            pltpu.SemaphoreType.DMA((_NB,)),
            pltpu.SemaphoreType.DMA((_NB,)),
            pltpu.SemaphoreType.DMA((_NB,)),
            pltpu.SemaphoreType.DMA((_NB,)),
        ],
    )(idx, vals, w_neigh)


_KB = 2048
_NKB = _VOC // _KB


def _tc_body(x_ref, p_ref, d_ref, ws_ref, g_ref, b_ref, fw_ref, fb_ref,
             o_ref, acc_ref):
    k = pl.program_id(0)

    @pl.when(k == 0)
    def _():
        acc_ref[...] = jnp.zeros_like(acc_ref)

    neigh = p_ref[0] + p_ref[1]
    dsum = jnp.sum(d_ref[...], axis=0, keepdims=True)
    deg = jnp.maximum(dsum.reshape(_KB, 1), 1.0)
    h = jnp.maximum(ws_ref[...] + neigh / deg, 0.0)
    mu = jnp.mean(h, axis=1, keepdims=True)
    hc = h - mu
    var = jnp.mean(hc * hc, axis=1, keepdims=True)
    hn = hc * lax.rsqrt(var + _EPS) * g_ref[...] + b_ref[...]
    acc_ref[...] += jnp.dot(x_ref[...], hn,
                            preferred_element_type=jnp.float32)

    @pl.when(k == _NKB - 1)
    def _():
        o_ref[...] = jnp.dot(acc_ref[...], fw_ref[...],
                             preferred_element_type=jnp.float32) + fb_ref[...]


def _tc_call(x, p, dt, w_self, g, b, fw, fb):
    return pl.pallas_call(
        _tc_body,
        grid=(_NKB,),
        in_specs=[
            pl.BlockSpec((_B, _KB), lambda k: (0, k)),
            pl.BlockSpec((_NC, _KB, _HID), lambda k: (0, k, 0)),
            pl.BlockSpec((_NC, _KB), lambda k: (0, k)),
            pl.BlockSpec((_KB, _HID), lambda k: (k, 0)),
            pl.BlockSpec((1, _HID), lambda k: (0, 0)),
            pl.BlockSpec((1, _HID), lambda k: (0, 0)),
            pl.BlockSpec((_HID, _HID), lambda k: (0, 0)),
            pl.BlockSpec((1, _HID), lambda k: (0, 0)),
        ],
        out_specs=pl.BlockSpec((_B, _HID), lambda k: (0, 0)),
        out_shape=jax.ShapeDtypeStruct((_B, _HID), jnp.float32),
        scratch_shapes=[pltpu.VMEM((_B, _HID), jnp.float32)],
    )(x, p, dt, w_self, g, b, fw, fb)


def kernel(adj_indices, adj_values, X_dv, W_self, W_neigh, ln_gamma, ln_beta,
           fc_W, fc_b):
    nnz = adj_values.shape[0]
    pad = _NNZ_PAD - nnz
    zi = jnp.arange(pad, dtype=jnp.int32) % _VOC
    idx = jnp.concatenate([adj_indices, jnp.stack([zi, zi])], axis=1)
    vals = jnp.concatenate([adj_values, jnp.zeros((pad,), jnp.float32)])
    p, d = _sc_partials(idx, vals, W_neigh)
    return _tc_call(X_dv, p, d, W_self, ln_gamma.reshape(1, _HID),
                    ln_beta.reshape(1, _HID), fc_W, fc_b.reshape(1, _HID))


# speedup vs baseline: 1.6810x; 1.0178x over previous
import jax
import jax.numpy as jnp
from jax import lax
from jax.experimental import pallas as pl
from jax.experimental.pallas import tpu as pltpu
from jax.experimental.pallas import tpu_sc as plsc

_VOC = 16384
_HID = 64
_B = 1024
_NC = 2
_NS = 16
_NW = _NC * _NS
_MB = 128
_NMB = 68
_EW = _MB * _NMB
_NNZ_PAD = _NW * _EW
_RPT = _VOC // _NS
_EPS = 1e-5
_NB = 6
_PF = 3


def _sc_body(idx_hbm, vals_hbm, w_hbm, p_hbm, d_hbm,
             row_v, col_v, vals_v, rows_buf, zrow, acc_s, deg_s,
             sem_g, sem_s, sem_d, sem_t):
    c = lax.axis_index("c")
    s = lax.axis_index("s")
    w = c * _NS + s

    init_scope = jax.named_scope("sc_init")
    init_scope.__enter__()
    pltpu.sync_copy(vals_hbm.at[pl.ds(w * _EW, _EW)], vals_v)

    zv = jnp.zeros((16,), jnp.float32)

    def zb_body(i, carry):
        rows_buf[0, i // 4, pl.ds((i % 4) * 16, 16)] = zv
        return carry
    lax.fori_loop(0, _MB * _HID // 16, zb_body, 0)

    def zr_body(i, carry):
        zrow[pl.ds(i * 16, 16)] = zv
        return carry
    lax.fori_loop(0, _RPT // 16, zr_body, 0)

    base = s * _RPT
    for i in range(_RPT // _MB):
        pltpu.sync_copy(rows_buf.at[0], acc_s.at[pl.ds(base + i * _MB, _MB)])
    pltpu.sync_copy(zrow, deg_s.at[pl.ds(base, _RPT)])
    plsc.subcore_barrier()
    init_scope.__exit__(None, None, None)
    main_scope = jax.named_scope("sc_mainloop")
    main_scope.__enter__()

    def start_stage(j, t):
        off = w * _EW + j * _MB
        pltpu.async_copy(idx_hbm.at[0, pl.ds(off, _MB)], row_v.at[t],
                         sem_t.at[t])
        pltpu.async_copy(idx_hbm.at[1, pl.ds(off, _MB)], col_v.at[t],
                         sem_t.at[t])

    def wait_stage(t):
        for _ in range(2):
            pltpu.make_async_copy(idx_hbm.at[0, pl.ds(0, _MB)], row_v.at[t],
                                  sem_t.at[t]).wait()

    def start_gather(j, b):
        pltpu.async_copy(w_hbm.at[col_v.at[b]], rows_buf.at[b], sem_g.at[b])

    def wait_gather(b):
        pltpu.make_async_copy(w_hbm.at[col_v.at[0]], rows_buf.at[b],
                              sem_g.at[b]).wait()

    def start_scatter(j, b):
        pltpu.async_copy(rows_buf.at[b], acc_s.at[row_v.at[b]], sem_s.at[b],
                         add=True)
        pltpu.async_copy(vals_v.at[pl.ds(j * _MB, _MB)],
                         deg_s.at[row_v.at[b]], sem_d.at[b], add=True)

    def wait_scatter(b):
        pltpu.make_async_copy(rows_buf.at[b], acc_s.at[row_v.at[0]],
                              sem_s.at[b]).wait()
        pltpu.make_async_copy(vals_v.at[pl.ds(0, _MB)],
                              deg_s.at[row_v.at[0]], sem_d.at[b]).wait()

    def scale(j, b):
        jj = j * _MB

        def blk_body(eb, carry2):
            vv = vals_v[pl.ds(jj + eb * 16, 16)]
            for l in range(16):
                val = jnp.full((16,), vv[l], jnp.float32)
                e = eb * 16 + l
                for q in range(4):
                    sl = (b, e, pl.ds(q * 16, 16))
                    rows_buf[sl] = rows_buf[sl] * val
            return carry2
        lax.fori_loop(0, _MB // 16, blk_body, 0)

    def run_iter(j, drain, stage, gather):
        b = j % _NB
        if drain:
            wait_scatter((j + _PF + 1) % _NB)
        if stage:
            start_stage(j + _PF + 1, (j + _PF + 1) % _NB)
        if gather:
            wait_stage((j + _PF) % _NB)
            start_gather(j + _PF, (j + _PF) % _NB)
        wait_gather(b)
        scale(j, b)
        start_scatter(j, b)

    for j in range(_PF + 1):
        start_stage(j, j)
    for j in range(_PF):
        wait_stage(j)
        start_gather(j, j)
    for j in range(_NB):
        run_iter(j, drain=(j >= _NB - _PF - 1), stage=True, gather=True)

    def mb_body(j, carry):
        run_iter(j, drain=True, stage=True, gather=True)
        return carry
    lax.fori_loop(_NB, _NMB - _PF - 1, mb_body, 0)

    for j in range(_NMB - _PF - 1, _NMB):
        run_iter(j, drain=True, stage=(j + _PF + 1 < _NMB),
                 gather=(j + _PF < _NMB))
    for j in range(_NMB - _NB + _PF + 1, _NMB):
        wait_scatter(j % _NB)

    main_scope.__exit__(None, None, None)
    out_scope = jax.named_scope("sc_readout")
    out_scope.__enter__()
    plsc.subcore_barrier()
    pltpu.sync_copy(acc_s.at[pl.ds(base, _RPT)],
                    p_hbm.at[c, pl.ds(base, _RPT)])
    pltpu.sync_copy(deg_s.at[pl.ds(base, _RPT)],
                    d_hbm.at[c, pl.ds(base, _RPT)])
    out_scope.__exit__(None, None, None)


def _sc_partials(idx, vals, w_neigh):
    mesh = plsc.VectorSubcoreMesh(core_axis_name="c", subcore_axis_name="s")
    return pl.kernel(
        _sc_body,
        out_type=[
            jax.ShapeDtypeStruct((_NC, _VOC, _HID), jnp.float32),
            jax.ShapeDtypeStruct((_NC, _VOC), jnp.float32),
        ],
        mesh=mesh,
        compiler_params=pltpu.CompilerParams(use_tc_tiling_on_sc=False),
        scratch_types=[
            pltpu.VMEM((_NB, _MB), jnp.int32),
            pltpu.VMEM((_NB, _MB), jnp.int32),
            pltpu.VMEM((_EW,), jnp.float32),
            pltpu.VMEM((_NB, _MB, _HID), jnp.float32),
            pltpu.VMEM((_RPT,), jnp.float32),
            pltpu.VMEM_SHARED((_VOC, _HID), jnp.float32),
            pltpu.VMEM_SHARED((_VOC,), jnp.float32),
            pltpu.SemaphoreType.DMA((_NB,)),
            pltpu.SemaphoreType.DMA((_NB,)),
            pltpu.SemaphoreType.DMA((_NB,)),
            pltpu.SemaphoreType.DMA((_NB,)),
        ],
    )(idx, vals, w_neigh)


_KB = 2048
_NKB = _VOC // _KB


def _tc_body(x_ref, p_ref, d_ref, ws_ref, g_ref, b_ref, fw_ref, fb_ref,
             o_ref, acc_ref):
    k = pl.program_id(0)

    @pl.when(k == 0)
    def _():
        acc_ref[...] = jnp.zeros_like(acc_ref)

    neigh = p_ref[0] + p_ref[1]
    dsum = jnp.sum(d_ref[...], axis=0, keepdims=True)
    deg = jnp.maximum(dsum.reshape(_KB, 1), 1.0)
    h = jnp.maximum(ws_ref[...] + neigh / deg, 0.0)
    mu = jnp.mean(h, axis=1, keepdims=True)
    hc = h - mu
    var = jnp.mean(hc * hc, axis=1, keepdims=True)
    hn = hc * lax.rsqrt(var + _EPS) * g_ref[...] + b_ref[...]
    acc_ref[...] += jnp.dot(x_ref[...], hn.astype(jnp.bfloat16),
                            preferred_element_type=jnp.float32)

    @pl.when(k == _NKB - 1)
    def _():
        o_ref[...] = jnp.dot(acc_ref[...], fw_ref[...],
                             preferred_element_type=jnp.float32) + fb_ref[...]


def _tc_call(x, p, dt, w_self, g, b, fw, fb):
    return pl.pallas_call(
        _tc_body,
        grid=(_NKB,),
        in_specs=[
            pl.BlockSpec((_B, _KB), lambda k: (0, k)),
            pl.BlockSpec((_NC, _KB, _HID), lambda k: (0, k, 0)),
            pl.BlockSpec((_NC, _KB), lambda k: (0, k)),
            pl.BlockSpec((_KB, _HID), lambda k: (k, 0)),
            pl.BlockSpec((1, _HID), lambda k: (0, 0)),
            pl.BlockSpec((1, _HID), lambda k: (0, 0)),
            pl.BlockSpec((_HID, _HID), lambda k: (0, 0)),
            pl.BlockSpec((1, _HID), lambda k: (0, 0)),
        ],
        out_specs=pl.BlockSpec((_B, _HID), lambda k: (0, 0)),
        out_shape=jax.ShapeDtypeStruct((_B, _HID), jnp.float32),
        scratch_shapes=[pltpu.VMEM((_B, _HID), jnp.float32)],
    )(x, p, dt, w_self, g, b, fw, fb)


def kernel(adj_indices, adj_values, X_dv, W_self, W_neigh, ln_gamma, ln_beta,
           fc_W, fc_b):
    nnz = adj_values.shape[0]
    pad = _NNZ_PAD - nnz
    zi = jnp.arange(pad, dtype=jnp.int32) % _VOC
    idx = jnp.concatenate([adj_indices, jnp.stack([zi, zi])], axis=1)
    vals = jnp.concatenate([adj_values, jnp.zeros((pad,), jnp.float32)])
    p, d = _sc_partials(idx, vals, W_neigh)
    return _tc_call(X_dv.astype(jnp.bfloat16), p, d, W_self, ln_gamma.reshape(1, _HID),
                    ln_beta.reshape(1, _HID), fc_W, fc_b.reshape(1, _HID))
